# baseline (speedup 1.0000x reference)
import jax
import jax.numpy as jnp
from jax import lax
from jax.experimental import pallas as pl
from jax.experimental.pallas import tpu as pltpu

P = 40
DIM = 3
H = 128
E = P * P
N_LAYERS = 4
GPB = 8
_NW = 11


def _silu(x):
    half = 0.5 * x
    return half * (jnp.tanh(half) + 1.0)


def _egnn_kernel(*refs):
    x0_ref, t_ref, we_ref, be_ref = refs[:4]
    out_ref = refs[-1]
    f32 = jnp.float32

    mi = lax.broadcasted_iota(jnp.int32, (P, P, 1), 0)
    mj = lax.broadcasted_iota(jnp.int32, (P, P, 1), 1)
    offdiag = (mi != mj).astype(f32)

    def md(a, b):
        return jnp.dot(a, b, preferred_element_type=f32)

    def pair_diff(y):
        return (jnp.broadcast_to(y[:, None, :], (P, P, DIM))
                - jnp.broadcast_to(y[None, :, :], (P, P, DIM))).reshape(E, DIM)

    t = t_ref[0, 0]
    h0_row = t * we_ref[...] + be_ref[...]

    for g in range(GPB):
        x0 = x0_ref[g]

        d0 = pair_diff(x0)
        ea = jnp.sum(d0 * d0, axis=1, keepdims=True)

        h = jnp.broadcast_to(h0_row, (P, H))
        x = x0

        for l in range(N_LAYERS):
            (w1_r, w2_r, wc1_r, wc2_r, wn1_r, wn2_r,
             b1_r, b2_r, bc1_r, bn1_r, bn2_r) = refs[4 + l * _NW:
                                                     4 + (l + 1) * _NW]

            d = pair_diff(x)
            radial = jnp.sum(d * d, axis=1, keepdims=True)
            f = 1.0 / (jnp.sqrt(radial + 1e-8) + 1.0)

            u = md(h, w1_r[0:H]) + b1_r[...]
            v = md(h, w1_r[H:2 * H])
            feat = jnp.concatenate([radial, ea], axis=1)
            pre = (jnp.broadcast_to(u[:, None, :], (P, P, H)).reshape(E, H)
                   + jnp.broadcast_to(v[None, :, :], (P, P, H)).reshape(E, H)
                   + md(feat, w1_r[2 * H:2 * H + 2]))
            m1 = _silu(pre)
            m = _silu(md(m1, w2_r[...]) + b2_r[...])

            q = _silu(md(m, wc1_r[...]) + bc1_r[...])
            p = md(q, wc2_r[...])
            x = x + jnp.sum((d * (f * p)).reshape(P, P, DIM), axis=1)
            agg = jnp.sum(m.reshape(P, P, H) * offdiag, axis=1)

            hh = jnp.concatenate([h, agg], axis=1)
            h = (h + md(_silu(md(hh, wn1_r[...]) + bn1_r[...]), wn2_r[...])
                 + bn2_r[...])

        vel = x - x0
        vel = vel - jnp.mean(vel, axis=0, keepdims=True)
        out_ref[g] = vel


@jax.jit
def kernel(t, xs, params, row, col):
    del row, col
    n_batch = xs.shape[0]
    x0 = xs.reshape(n_batch, P, DIM)
    t2d = t.reshape(1, 1)
    We, be = params['emb']

    args = [x0, t2d, We.reshape(1, H), be.reshape(1, H)]
    for lp in params['layers']:
        W1, b1 = lp['edge1']
        W2, b2 = lp['edge2']
        Wc1, bc1 = lp['coord1']
        Wn1, bn1 = lp['node1']
        Wn2, bn2 = lp['node2']
        args += [W1, W2, Wc1, lp['coord2'], Wn1, Wn2,
                 b1.reshape(1, H), b2.reshape(1, H), bc1.reshape(1, H),
                 bn1.reshape(1, H), bn2.reshape(1, H)]

    full = lambda shape: pl.BlockSpec(shape, lambda b: (0,) * len(shape))
    in_specs = [pl.BlockSpec((GPB, P, DIM), lambda b: (b, 0, 0))]
    in_specs += [full(a.shape) for a in args[1:]]
    out = pl.pallas_call(
        _egnn_kernel,
        grid=(n_batch // GPB,),
        in_specs=in_specs,
        out_specs=pl.BlockSpec((GPB, P, DIM), lambda b: (b, 0, 0)),
        out_shape=jax.ShapeDtypeStruct((n_batch, P, DIM), jnp.float32),
        compiler_params=pltpu.CompilerParams(
            dimension_semantics=("arbitrary",)),
    )(*args)
    return out.reshape(n_batch, P * DIM)


# speedup vs baseline: 1.3597x; 1.3597x over previous
import jax
import jax.numpy as jnp
from jax import lax
from jax.experimental import pallas as pl
from jax.experimental.pallas import tpu as pltpu

P = 40
DIM = 3
H = 128
E = P * P
N_LAYERS = 4
GPB = 2
_NW = 11


def _silu(x):
    half = 0.5 * x
    return half * (jnp.tanh(half) + 1.0)


def _egnn_kernel(*refs):
    x0_ref, t_ref, we_ref, be_ref = refs[:4]
    out_ref = refs[-1]
    f32 = jnp.float32

    mi = lax.broadcasted_iota(jnp.int32, (P, P, 1), 0)
    mj = lax.broadcasted_iota(jnp.int32, (P, P, 1), 1)
    offdiag = (mi != mj).astype(f32)

    def md(a, b):
        return jnp.dot(a, b, preferred_element_type=f32)

    def pair_diff(y):
        return (jnp.broadcast_to(y[:, None, :], (P, P, DIM))
                - jnp.broadcast_to(y[None, :, :], (P, P, DIM))).reshape(E, DIM)

    t = t_ref[0, 0]
    h0_row = t * we_ref[...] + be_ref[...]

    for g in range(GPB):
        x0 = x0_ref[g]

        d0 = pair_diff(x0)
        ea = jnp.sum(d0 * d0, axis=1, keepdims=True)

        h = jnp.broadcast_to(h0_row, (P, H))
        x = x0

        for l in range(N_LAYERS):
            (w1_r, w2_r, wc1_r, wc2_r, wn1_r, wn2_r,
             b1_r, b2_r, bc1_r, bn1_r, bn2_r) = refs[4 + l * _NW:
                                                     4 + (l + 1) * _NW]

            d = pair_diff(x)
            radial = jnp.sum(d * d, axis=1, keepdims=True)
            f = 1.0 / (jnp.sqrt(radial + 1e-8) + 1.0)

            u = md(h, w1_r[0:H]) + b1_r[...]
            v = md(h, w1_r[H:2 * H])
            feat = jnp.concatenate([radial, ea], axis=1)
            pre = (jnp.broadcast_to(u[:, None, :], (P, P, H)).reshape(E, H)
                   + jnp.broadcast_to(v[None, :, :], (P, P, H)).reshape(E, H)
                   + md(feat, w1_r[2 * H:2 * H + 2]))
            m1 = _silu(pre)
            m = _silu(md(m1, w2_r[...]) + b2_r[...])

            q = _silu(md(m, wc1_r[...]) + bc1_r[...])
            p = md(q, wc2_r[...])
            x = x + jnp.sum((d * (f * p)).reshape(P, P, DIM), axis=1)
            agg = jnp.sum(m.reshape(P, P, H) * offdiag, axis=1)

            hh = jnp.concatenate([h, agg], axis=1)
            h = (h + md(_silu(md(hh, wn1_r[...]) + bn1_r[...]), wn2_r[...])
                 + bn2_r[...])

        vel = x - x0
        vel = vel - jnp.mean(vel, axis=0, keepdims=True)
        out_ref[g] = vel


@jax.jit
def kernel(t, xs, params, row, col):
    del row, col
    n_batch = xs.shape[0]
    x0 = xs.reshape(n_batch, P, DIM)
    t2d = t.reshape(1, 1)
    We, be = params['emb']

    args = [x0, t2d, We.reshape(1, H), be.reshape(1, H)]
    for lp in params['layers']:
        W1, b1 = lp['edge1']
        W2, b2 = lp['edge2']
        Wc1, bc1 = lp['coord1']
        Wn1, bn1 = lp['node1']
        Wn2, bn2 = lp['node2']
        args += [W1, W2, Wc1, lp['coord2'], Wn1, Wn2,
                 b1.reshape(1, H), b2.reshape(1, H), bc1.reshape(1, H),
                 bn1.reshape(1, H), bn2.reshape(1, H)]

    full = lambda shape: pl.BlockSpec(shape, lambda b: (0,) * len(shape))
    in_specs = [pl.BlockSpec((GPB, P, DIM), lambda b: (b, 0, 0))]
    in_specs += [full(a.shape) for a in args[1:]]
    out = pl.pallas_call(
        _egnn_kernel,
        grid=(n_batch // GPB,),
        in_specs=in_specs,
        out_specs=pl.BlockSpec((GPB, P, DIM), lambda b: (b, 0, 0)),
        out_shape=jax.ShapeDtypeStruct((n_batch, P, DIM), jnp.float32),
        compiler_params=pltpu.CompilerParams(
            dimension_semantics=("arbitrary",)),
    )(*args)
    return out.reshape(n_batch, P * DIM)


# speedup vs baseline: 1.3780x; 1.0134x over previous
import jax
import jax.numpy as jnp
from jax import lax
from jax.experimental import pallas as pl
from jax.experimental.pallas import tpu as pltpu

P = 40
DIM = 3
H = 128
E = P * P
N_LAYERS = 4
GPB = 4
_NW = 11


def _silu(x):
    half = 0.5 * x
    return half * (jnp.tanh(half) + 1.0)


def _egnn_kernel(*refs):
    x0_ref, t_ref, we_ref, be_ref = refs[:4]
    out_ref = refs[-1]
    f32 = jnp.float32

    mi = lax.broadcasted_iota(jnp.int32, (P, P, 1), 0)
    mj = lax.broadcasted_iota(jnp.int32, (P, P, 1), 1)
    offdiag = (mi != mj).astype(f32)

    def md(a, b):
        return jnp.dot(a, b, preferred_element_type=f32)

    def pair_diff(y):
        return (jnp.broadcast_to(y[:, None, :], (P, P, DIM))
                - jnp.broadcast_to(y[None, :, :], (P, P, DIM))).reshape(E, DIM)

    t = t_ref[0, 0]
    h0_row = t * we_ref[...] + be_ref[...]

    for g in range(GPB):
        x0 = x0_ref[g]

        d0 = pair_diff(x0)
        ea = jnp.sum(d0 * d0, axis=1, keepdims=True)

        h = jnp.broadcast_to(h0_row, (P, H))
        x = x0

        for l in range(N_LAYERS):
            (w1_r, w2_r, wc1_r, wc2_r, wn1_r, wn2_r,
             b1_r, b2_r, bc1_r, bn1_r, bn2_r) = refs[4 + l * _NW:
                                                     4 + (l + 1) * _NW]

            d = pair_diff(x)
            radial = jnp.sum(d * d, axis=1, keepdims=True)
            f = 1.0 / (jnp.sqrt(radial + 1e-8) + 1.0)

            u = md(h, w1_r[0:H]) + b1_r[...]
            v = md(h, w1_r[H:2 * H])
            feat = jnp.concatenate([radial, ea], axis=1)
            pre = (jnp.broadcast_to(u[:, None, :], (P, P, H)).reshape(E, H)
                   + jnp.broadcast_to(v[None, :, :], (P, P, H)).reshape(E, H)
                   + md(feat, w1_r[2 * H:2 * H + 2]))
            m1 = _silu(pre)
            m = _silu(md(m1, w2_r[...]) + b2_r[...])

            q = _silu(md(m, wc1_r[...]) + bc1_r[...])
            p = md(q, wc2_r[...])
            x = x + jnp.sum((d * (f * p)).reshape(P, P, DIM), axis=1)
            agg = jnp.sum(m.reshape(P, P, H) * offdiag, axis=1)

            hh = jnp.concatenate([h, agg], axis=1)
            h = (h + md(_silu(md(hh, wn1_r[...]) + bn1_r[...]), wn2_r[...])
                 + bn2_r[...])

        vel = x - x0
        vel = vel - jnp.mean(vel, axis=0, keepdims=True)
        out_ref[g] = vel


@jax.jit
def kernel(t, xs, params, row, col):
    del row, col
    n_batch = xs.shape[0]
    x0 = xs.reshape(n_batch, P, DIM)
    t2d = t.reshape(1, 1)
    We, be = params['emb']

    args = [x0, t2d, We.reshape(1, H), be.reshape(1, H)]
    for lp in params['layers']:
        W1, b1 = lp['edge1']
        W2, b2 = lp['edge2']
        Wc1, bc1 = lp['coord1']
        Wn1, bn1 = lp['node1']
        Wn2, bn2 = lp['node2']
        args += [W1, W2, Wc1, lp['coord2'], Wn1, Wn2,
                 b1.reshape(1, H), b2.reshape(1, H), bc1.reshape(1, H),
                 bn1.reshape(1, H), bn2.reshape(1, H)]

    full = lambda shape: pl.BlockSpec(shape, lambda b: (0,) * len(shape))
    in_specs = [pl.BlockSpec((GPB, P, DIM), lambda b: (b, 0, 0))]
    in_specs += [full(a.shape) for a in args[1:]]
    out = pl.pallas_call(
        _egnn_kernel,
        grid=(n_batch // GPB,),
        in_specs=in_specs,
        out_specs=pl.BlockSpec((GPB, P, DIM), lambda b: (b, 0, 0)),
        out_shape=jax.ShapeDtypeStruct((n_batch, P, DIM), jnp.float32),
        compiler_params=pltpu.CompilerParams(
            dimension_semantics=("arbitrary",)),
    )(*args)
    return out.reshape(n_batch, P * DIM)


# speedup vs baseline: 1.4030x; 1.0181x over previous
import jax
import jax.numpy as jnp
from jax import lax
from jax.experimental import pallas as pl
from jax.experimental.pallas import tpu as pltpu

P = 40
DIM = 3
H = 128
E = P * P
N_LAYERS = 4
GPB = 4
_NW = 11


def _silu(x):
    half = 0.5 * x
    return half * (jnp.tanh(half) + 1.0)


def _egnn_kernel(*refs):
    x0_ref, t_ref, we_ref, be_ref = refs[:4]
    out_ref = refs[-1]
    f32 = jnp.float32

    mi = lax.broadcasted_iota(jnp.int32, (P, P, 1), 0)
    mj = lax.broadcasted_iota(jnp.int32, (P, P, 1), 1)
    offdiag = (mi != mj).astype(f32)

    def md(a, b):
        return jnp.dot(a, b, preferred_element_type=f32)

    def pair_diff(y):
        return (jnp.broadcast_to(y[:, None, :], (P, P, DIM))
                - jnp.broadcast_to(y[None, :, :], (P, P, DIM))).reshape(E, DIM)

    t = t_ref[0, 0]
    h0_row = t * we_ref[...] + be_ref[...]

    for g in range(GPB):
        x0 = x0_ref[g]

        d0 = pair_diff(x0)
        ea = jnp.sum(d0 * d0, axis=1, keepdims=True)

        h = jnp.broadcast_to(h0_row, (P, H))
        x = x0

        for l in range(N_LAYERS):
            (w1_r, w2_r, wc1_r, wc2_r, wn1_r, wn2_r,
             b1_r, b2_r, bc1_r, bn1_r, bn2_r) = refs[4 + l * _NW:
                                                     4 + (l + 1) * _NW]

            d = pair_diff(x)
            radial = jnp.sum(d * d, axis=1, keepdims=True)
            cd = d / (jnp.sqrt(radial + 1e-8) + 1.0)

            u = md(h, w1_r[0:H]) + b1_r[...]
            v = md(h, w1_r[H:2 * H])
            feat = jnp.concatenate([radial, ea], axis=1)
            pre = (jnp.broadcast_to(u[:, None, :], (P, P, H)).reshape(E, H)
                   + jnp.broadcast_to(v[None, :, :], (P, P, H)).reshape(E, H)
                   + md(feat, w1_r[2 * H:2 * H + 2]))
            m1 = _silu(pre)
            m = _silu(md(m1, w2_r[...]) + b2_r[...])

            q = _silu(md(m, wc1_r[...]) + bc1_r[...])
            p = md(q, wc2_r[...])
            x = x + jnp.sum((cd * p).reshape(P, P, DIM), axis=1)
            agg = jnp.sum(m.reshape(P, P, H) * offdiag, axis=1)

            hh = jnp.concatenate([h, agg], axis=1)
            h = (h + md(_silu(md(hh, wn1_r[...]) + bn1_r[...]), wn2_r[...])
                 + bn2_r[...])

        vel = x - x0
        vel = vel - jnp.mean(vel, axis=0, keepdims=True)
        out_ref[g] = vel


@jax.jit
def kernel(t, xs, params, row, col):
    del row, col
    n_batch = xs.shape[0]
    x0 = xs.reshape(n_batch, P, DIM)
    t2d = t.reshape(1, 1)
    We, be = params['emb']

    args = [x0, t2d, We.reshape(1, H), be.reshape(1, H)]
    for lp in params['layers']:
        W1, b1 = lp['edge1']
        W2, b2 = lp['edge2']
        Wc1, bc1 = lp['coord1']
        Wn1, bn1 = lp['node1']
        Wn2, bn2 = lp['node2']
        args += [W1, W2, Wc1, lp['coord2'], Wn1, Wn2,
                 b1.reshape(1, H), b2.reshape(1, H), bc1.reshape(1, H),
                 bn1.reshape(1, H), bn2.reshape(1, H)]

    full = lambda shape: pl.BlockSpec(shape, lambda b: (0,) * len(shape))
    in_specs = [pl.BlockSpec((GPB, P, DIM), lambda b: (b, 0, 0))]
    in_specs += [full(a.shape) for a in args[1:]]
    out = pl.pallas_call(
        _egnn_kernel,
        grid=(n_batch // GPB,),
        in_specs=in_specs,
        out_specs=pl.BlockSpec((GPB, P, DIM), lambda b: (b, 0, 0)),
        out_shape=jax.ShapeDtypeStruct((n_batch, P, DIM), jnp.float32),
        compiler_params=pltpu.CompilerParams(
            dimension_semantics=("arbitrary",)),
    )(*args)
    return out.reshape(n_batch, P * DIM)
